# baseline (speedup 1.0000x reference)
import functools

import jax
import jax.numpy as jnp
from jax.experimental import pallas as pl
from jax.experimental.pallas import tpu as pltpu
from jax.experimental.pallas import tpu_sc as plsc

N = 100
T = 64
F = 128
H = 256
EMB = 64
FLAT_IN = 32
FLAT_OUT = 64
GNN_HID = 256
GNN_OUT = 128
E = 800
NP = 128
L = 16


def _sigmoid(x):
    return jnp.tanh(x * 0.5) * 0.5 + 0.5


CH = 80


def _adj_body(src_hbm, dst_hbm, eye_hbm, zeros_hbm, out_hbm,
              src_v, dst_v, buf_v, a_sh):
    cid = jax.lax.axis_index("c")
    sid = jax.lax.axis_index("s")

    @pl.when(jnp.logical_and(cid == 0, sid == 0))
    def _():
        pltpu.sync_copy(zeros_hbm, a_sh)
        for c in range(E // CH):
            sl = pl.ds(c * CH, CH)
            pltpu.sync_copy(src_hbm.at[sl], src_v)
            pltpu.sync_copy(dst_hbm.at[sl], dst_v)
            pltpu.sync_copy(eye_hbm.at[src_v], buf_v)
            pltpu.sync_copy(buf_v, a_sh.at[dst_v], add=True)
        pltpu.sync_copy(a_sh, out_hbm)


def _adjacency_sc(edge_index, eye, zeros):
    f32 = jnp.float32
    k = pl.kernel(
        _adj_body,
        out_type=jax.ShapeDtypeStruct((NP, NP), f32),
        mesh=plsc.VectorSubcoreMesh(core_axis_name="c", subcore_axis_name="s"),
        scratch_types=[
            pltpu.VMEM((CH,), jnp.int32),
            pltpu.VMEM((CH,), jnp.int32),
            pltpu.VMEM((CH, NP), f32),
            pltpu.VMEM_SHARED((NP, NP), f32),
        ],
    )
    return k(edge_index[0], edge_index[1], eye, zeros)


def _gru_body(xT_ref, WihT_ref, WhhT_ref, brz_ref, bihn_ref, bhhn_ref,
              h_out_ref, gi_ref):
    f32 = jnp.float32
    bf16 = jnp.bfloat16
    WhhT = WhhT_ref[...]
    brz = brz_ref[...]
    bihn = bihn_ref[...]
    bhhn = bhhn_ref[...]

    gi_ref[...] = jnp.dot(xT_ref[...], WihT_ref[...],
                          preferred_element_type=f32).astype(bf16)

    def substep(t, h):
        gi = gi_ref[pl.ds(t * NP, NP), :].astype(f32)
        gh = jnp.dot(h.astype(bf16), WhhT, preferred_element_type=f32)
        rz = _sigmoid(gi[:, 0:2 * H] + gh[:, 0:2 * H] + brz)
        r = rz[:, 0:H]
        z = rz[:, H:2 * H]
        n = jnp.tanh(gi[:, 2 * H:3 * H] + bihn + r * (gh[:, 2 * H:3 * H] + bhhn))
        return n + z * (h - n)

    def step(i, h):
        h = substep(2 * i, h)
        return substep(2 * i + 1, h)

    h_out_ref[...] = jax.lax.fori_loop(0, T // 2, step,
                                       jnp.zeros((NP, H), f32))


def _gnn_body(h_ref, A_ref, flat_ref, emb_ref,
              Wf_ref, bf_ref, Wl1_ref, bl1_ref, Wr1_ref,
              Wl2_ref, bl2_ref, Wr2_ref, Wo_ref, bo_ref,
              out_ref):
    f32 = jnp.float32
    bf16 = jnp.bfloat16
    h = h_ref[...]
    A = A_ref[...]

    flat_enc = (
        jnp.dot(flat_ref[...], Wf_ref[...], preferred_element_type=f32)
        + bf_ref[...]
    )
    gnn_in = jnp.concatenate([h, flat_enc, emb_ref[...]], axis=1)

    cnt = jnp.sum(A, axis=1, keepdims=True)
    denom = jnp.maximum(cnt, 1.0)
    Ab = A.astype(bf16)

    mean1 = (jnp.dot(Ab, gnn_in.astype(bf16), preferred_element_type=f32)
             / denom)
    h1 = jax.nn.relu(
        jnp.dot(mean1.astype(bf16), Wl1_ref[...], preferred_element_type=f32)
        + bl1_ref[...]
        + jnp.dot(gnn_in.astype(bf16), Wr1_ref[...], preferred_element_type=f32)
    )
    mean2 = (jnp.dot(Ab, h1.astype(bf16), preferred_element_type=f32)
             / denom)
    g2 = (
        jnp.dot(mean2.astype(bf16), Wl2_ref[...], preferred_element_type=f32)
        + bl2_ref[...]
        + jnp.dot(h1.astype(bf16), Wr2_ref[...], preferred_element_type=f32)
    )

    cat = jnp.concatenate([g2, h], axis=1)
    logits = jnp.dot(cat, Wo_ref[...], preferred_element_type=f32) + bo_ref[...]
    out_ref[...] = _sigmoid(logits)


def kernel(node_feat, flat, edge_index, W_ih, W_hh, b_ih, b_hh, emb,
           Wf, bf, Wl1, bl1, Wr1, Wl2, bl2, Wr2, Wo, bo):
    f32 = jnp.float32
    bf16 = jnp.bfloat16
    xT = jnp.transpose(node_feat, (1, 0, 2))
    xT = jnp.pad(xT, ((0, 0), (0, NP - N), (0, 0)))
    xT = xT.reshape(T * NP, F).astype(bf16)
    flat_p = jnp.pad(flat, ((0, NP - N), (0, 0)))
    emb_p = jnp.pad(emb, ((0, NP - N), (0, 0)))
    brz = (b_ih[:2 * H] + b_hh[:2 * H]).reshape(1, -1)
    bihn = b_ih[2 * H:].reshape(1, -1)
    bhhn = b_hh[2 * H:].reshape(1, -1)

    A = _adjacency_sc(edge_index, jnp.eye(NP, dtype=f32),
                      jnp.zeros((NP, NP), f32))

    h = pl.pallas_call(
        _gru_body,
        out_shape=jax.ShapeDtypeStruct((NP, H), f32),
        scratch_shapes=[pltpu.VMEM((T * NP, 3 * H), bf16)],
    )(
        xT, W_ih.T.astype(bf16), W_hh.T.astype(bf16),
        brz, bihn, bhhn,
    )

    out = pl.pallas_call(
        _gnn_body,
        out_shape=jax.ShapeDtypeStruct((NP, 1), f32),
    )(
        h, A, flat_p, emb_p,
        Wf, bf.reshape(1, -1),
        Wl1.astype(bf16), bl1.reshape(1, -1), Wr1.astype(bf16),
        Wl2.astype(bf16), bl2.reshape(1, -1), Wr2.astype(bf16),
        Wo, bo.reshape(1, 1),
    )
    return out[:N, 0]


# speedup vs baseline: 1.0048x; 1.0048x over previous
import functools

import jax
import jax.numpy as jnp
from jax.experimental import pallas as pl
from jax.experimental.pallas import tpu as pltpu
from jax.experimental.pallas import tpu_sc as plsc

N = 100
T = 64
F = 128
H = 256
EMB = 64
FLAT_IN = 32
FLAT_OUT = 64
GNN_HID = 256
GNN_OUT = 128
E = 800
NP = 128
L = 16


def _sigmoid(x):
    return jnp.tanh(x * 0.5) * 0.5 + 0.5


CH = 80


NCHUNK = E // CH
RPS = NP // 16


def _adj_body(src_hbm, dst_hbm, eye_hbm, zeros_hbm, out_hbm,
              src_v, dst_v, buf_v, a_sh):
    cid = jax.lax.axis_index("c")
    sid = jax.lax.axis_index("s")
    rows = pl.ds(sid * RPS, RPS)

    @pl.when(cid == 0)
    def _():
        pltpu.sync_copy(zeros_hbm.at[rows], a_sh.at[rows])

    plsc.subcore_barrier()

    @pl.when(jnp.logical_and(cid == 0, sid < NCHUNK))
    def _():
        sl = pl.ds(sid * CH, CH)
        pltpu.sync_copy(src_hbm.at[sl], src_v)
        pltpu.sync_copy(dst_hbm.at[sl], dst_v)
        pltpu.sync_copy(eye_hbm.at[src_v], buf_v)
        pltpu.sync_copy(buf_v, a_sh.at[dst_v], add=True)

    plsc.subcore_barrier()

    @pl.when(cid == 0)
    def _():
        pltpu.sync_copy(a_sh.at[rows], out_hbm.at[rows])


def _adjacency_sc(edge_index, eye, zeros):
    f32 = jnp.float32
    k = pl.kernel(
        _adj_body,
        out_type=jax.ShapeDtypeStruct((NP, NP), f32),
        mesh=plsc.VectorSubcoreMesh(core_axis_name="c", subcore_axis_name="s"),
        scratch_types=[
            pltpu.VMEM((CH,), jnp.int32),
            pltpu.VMEM((CH,), jnp.int32),
            pltpu.VMEM((CH, NP), f32),
            pltpu.VMEM_SHARED((NP, NP), f32),
        ],
    )
    return k(edge_index[0], edge_index[1], eye, zeros)


def _gru_body(xT_ref, WihT_ref, WhhT_ref, brz_ref, bihn_ref, bhhn_ref,
              h_out_ref, gi_ref):
    f32 = jnp.float32
    bf16 = jnp.bfloat16
    WhhT = WhhT_ref[...]
    brz = brz_ref[...]
    bihn = bihn_ref[...]
    bhhn = bhhn_ref[...]

    gi_ref[...] = jnp.dot(xT_ref[...], WihT_ref[...],
                          preferred_element_type=f32).astype(bf16)

    def substep(t, h):
        gi = gi_ref[pl.ds(t * NP, NP), :].astype(f32)
        gh = jnp.dot(h.astype(bf16), WhhT, preferred_element_type=f32)
        rz = _sigmoid(gi[:, 0:2 * H] + gh[:, 0:2 * H] + brz)
        r = rz[:, 0:H]
        z = rz[:, H:2 * H]
        n = jnp.tanh(gi[:, 2 * H:3 * H] + bihn + r * (gh[:, 2 * H:3 * H] + bhhn))
        return n + z * (h - n)

    def step(i, h):
        h = substep(2 * i, h)
        return substep(2 * i + 1, h)

    h_out_ref[...] = jax.lax.fori_loop(0, T // 2, step,
                                       jnp.zeros((NP, H), f32))


def _gnn_body(h_ref, A_ref, flat_ref, emb_ref,
              Wf_ref, bf_ref, Wl1_ref, bl1_ref, Wr1_ref,
              Wl2_ref, bl2_ref, Wr2_ref, Wo_ref, bo_ref,
              out_ref):
    f32 = jnp.float32
    bf16 = jnp.bfloat16
    h = h_ref[...]
    A = A_ref[...]

    flat_enc = (
        jnp.dot(flat_ref[...], Wf_ref[...], preferred_element_type=f32)
        + bf_ref[...]
    )
    gnn_in = jnp.concatenate([h, flat_enc, emb_ref[...]], axis=1)

    cnt = jnp.sum(A, axis=1, keepdims=True)
    denom = jnp.maximum(cnt, 1.0)
    Ab = A.astype(bf16)

    mean1 = (jnp.dot(Ab, gnn_in.astype(bf16), preferred_element_type=f32)
             / denom)
    h1 = jax.nn.relu(
        jnp.dot(mean1.astype(bf16), Wl1_ref[...], preferred_element_type=f32)
        + bl1_ref[...]
        + jnp.dot(gnn_in.astype(bf16), Wr1_ref[...], preferred_element_type=f32)
    )
    mean2 = (jnp.dot(Ab, h1.astype(bf16), preferred_element_type=f32)
             / denom)
    g2 = (
        jnp.dot(mean2.astype(bf16), Wl2_ref[...], preferred_element_type=f32)
        + bl2_ref[...]
        + jnp.dot(h1.astype(bf16), Wr2_ref[...], preferred_element_type=f32)
    )

    cat = jnp.concatenate([g2, h], axis=1)
    logits = jnp.dot(cat, Wo_ref[...], preferred_element_type=f32) + bo_ref[...]
    out_ref[...] = _sigmoid(logits)


def kernel(node_feat, flat, edge_index, W_ih, W_hh, b_ih, b_hh, emb,
           Wf, bf, Wl1, bl1, Wr1, Wl2, bl2, Wr2, Wo, bo):
    f32 = jnp.float32
    bf16 = jnp.bfloat16
    xT = jnp.transpose(node_feat, (1, 0, 2))
    xT = jnp.pad(xT, ((0, 0), (0, NP - N), (0, 0)))
    xT = xT.reshape(T * NP, F).astype(bf16)
    flat_p = jnp.pad(flat, ((0, NP - N), (0, 0)))
    emb_p = jnp.pad(emb, ((0, NP - N), (0, 0)))
    brz = (b_ih[:2 * H] + b_hh[:2 * H]).reshape(1, -1)
    bihn = b_ih[2 * H:].reshape(1, -1)
    bhhn = b_hh[2 * H:].reshape(1, -1)

    A = _adjacency_sc(edge_index, jnp.eye(NP, dtype=f32),
                      jnp.zeros((NP, NP), f32))

    h = pl.pallas_call(
        _gru_body,
        out_shape=jax.ShapeDtypeStruct((NP, H), f32),
        scratch_shapes=[pltpu.VMEM((T * NP, 3 * H), bf16)],
    )(
        xT, W_ih.T.astype(bf16), W_hh.T.astype(bf16),
        brz, bihn, bhhn,
    )

    out = pl.pallas_call(
        _gnn_body,
        out_shape=jax.ShapeDtypeStruct((NP, 1), f32),
    )(
        h, A, flat_p, emb_p,
        Wf, bf.reshape(1, -1),
        Wl1.astype(bf16), bl1.reshape(1, -1), Wr1.astype(bf16),
        Wl2.astype(bf16), bl2.reshape(1, -1), Wr2.astype(bf16),
        Wo, bo.reshape(1, 1),
    )
    return out[:N, 0]


# speedup vs baseline: 1.0325x; 1.0275x over previous
import functools

import jax
import jax.numpy as jnp
from jax.experimental import pallas as pl
from jax.experimental.pallas import tpu as pltpu
from jax.experimental.pallas import tpu_sc as plsc

N = 100
T = 64
F = 128
H = 256
EMB = 64
FLAT_IN = 32
FLAT_OUT = 64
GNN_HID = 256
GNN_OUT = 128
E = 800
NP = 128
L = 16


def _sigmoid(x):
    return jnp.tanh(x * 0.5) * 0.5 + 0.5


CH = 80


NCHUNK = E // CH
RPS = NP // 16


def _adj_body(src_hbm, dst_hbm, eye_hbm, zeros_hbm, out_hbm,
              src_v, dst_v, buf_v, a_sh):
    cid = jax.lax.axis_index("c")
    sid = jax.lax.axis_index("s")
    rows = pl.ds(sid * RPS, RPS)

    @pl.when(cid == 0)
    def _():
        pltpu.sync_copy(zeros_hbm.at[rows], a_sh.at[rows])

    plsc.subcore_barrier()

    @pl.when(jnp.logical_and(cid == 0, sid < NCHUNK))
    def _():
        sl = pl.ds(sid * CH, CH)
        pltpu.sync_copy(src_hbm.at[sl], src_v)
        pltpu.sync_copy(dst_hbm.at[sl], dst_v)
        pltpu.sync_copy(eye_hbm.at[src_v], buf_v)
        pltpu.sync_copy(buf_v, a_sh.at[dst_v], add=True)

    plsc.subcore_barrier()

    @pl.when(cid == 0)
    def _():
        pltpu.sync_copy(a_sh.at[rows], out_hbm.at[rows])


def _adjacency_sc(edge_index, eye, zeros):
    f32 = jnp.float32
    k = pl.kernel(
        _adj_body,
        out_type=jax.ShapeDtypeStruct((NP, NP), f32),
        mesh=plsc.VectorSubcoreMesh(core_axis_name="c", subcore_axis_name="s"),
        scratch_types=[
            pltpu.VMEM((CH,), jnp.int32),
            pltpu.VMEM((CH,), jnp.int32),
            pltpu.VMEM((CH, NP), f32),
            pltpu.VMEM_SHARED((NP, NP), f32),
        ],
    )
    return k(edge_index[0], edge_index[1], eye, zeros)


def _tc_body(xT_ref, WihT_ref, WhhT_ref, brz_ref, bihn_ref, bhhn_ref,
             A_ref, flat_ref, emb_ref,
             Wf_ref, bf_ref, Wl1_ref, bl1_ref, Wr1_ref,
             Wl2_ref, bl2_ref, Wr2_ref, Wo_ref, bo_ref,
             out_ref, gi_ref):
    f32 = jnp.float32
    bf16 = jnp.bfloat16
    WhhT = WhhT_ref[...]
    brz = brz_ref[...]
    bihn = bihn_ref[...]
    bhhn = bhhn_ref[...]

    gi_ref[...] = jnp.dot(xT_ref[...], WihT_ref[...],
                          preferred_element_type=f32).astype(bf16)

    def substep(t, h):
        gi = gi_ref[pl.ds(t * NP, NP), :].astype(f32)
        gh = jnp.dot(h.astype(bf16), WhhT, preferred_element_type=f32)
        rz = _sigmoid(gi[:, 0:2 * H] + gh[:, 0:2 * H] + brz)
        r = rz[:, 0:H]
        z = rz[:, H:2 * H]
        n = jnp.tanh(gi[:, 2 * H:3 * H] + bihn + r * (gh[:, 2 * H:3 * H] + bhhn))
        return n + z * (h - n)

    def step(i, h):
        h = substep(2 * i, h)
        return substep(2 * i + 1, h)

    h = jax.lax.fori_loop(0, T // 2, step, jnp.zeros((NP, H), f32))
    A = A_ref[...]

    flat_enc = (
        jnp.dot(flat_ref[...], Wf_ref[...], preferred_element_type=f32)
        + bf_ref[...]
    )
    gnn_in = jnp.concatenate([h, flat_enc, emb_ref[...]], axis=1)

    cnt = jnp.sum(A, axis=1, keepdims=True)
    denom = jnp.maximum(cnt, 1.0)
    Ab = A.astype(bf16)

    mean1 = (jnp.dot(Ab, gnn_in.astype(bf16), preferred_element_type=f32)
             / denom)
    h1 = jax.nn.relu(
        jnp.dot(mean1.astype(bf16), Wl1_ref[...], preferred_element_type=f32)
        + bl1_ref[...]
        + jnp.dot(gnn_in.astype(bf16), Wr1_ref[...], preferred_element_type=f32)
    )
    mean2 = (jnp.dot(Ab, h1.astype(bf16), preferred_element_type=f32)
             / denom)
    g2 = (
        jnp.dot(mean2.astype(bf16), Wl2_ref[...], preferred_element_type=f32)
        + bl2_ref[...]
        + jnp.dot(h1.astype(bf16), Wr2_ref[...], preferred_element_type=f32)
    )

    cat = jnp.concatenate([g2, h], axis=1)
    logits = jnp.dot(cat, Wo_ref[...], preferred_element_type=f32) + bo_ref[...]
    out_ref[...] = _sigmoid(logits)


def kernel(node_feat, flat, edge_index, W_ih, W_hh, b_ih, b_hh, emb,
           Wf, bf, Wl1, bl1, Wr1, Wl2, bl2, Wr2, Wo, bo):
    f32 = jnp.float32
    bf16 = jnp.bfloat16
    xT = jnp.transpose(node_feat, (1, 0, 2))
    xT = jnp.pad(xT, ((0, 0), (0, NP - N), (0, 0)))
    xT = xT.reshape(T * NP, F).astype(bf16)
    flat_p = jnp.pad(flat, ((0, NP - N), (0, 0)))
    emb_p = jnp.pad(emb, ((0, NP - N), (0, 0)))
    brz = (b_ih[:2 * H] + b_hh[:2 * H]).reshape(1, -1)
    bihn = b_ih[2 * H:].reshape(1, -1)
    bhhn = b_hh[2 * H:].reshape(1, -1)

    A = _adjacency_sc(edge_index, jnp.eye(NP, dtype=f32),
                      jnp.zeros((NP, NP), f32))

    out = pl.pallas_call(
        _tc_body,
        out_shape=jax.ShapeDtypeStruct((NP, 1), f32),
        scratch_shapes=[pltpu.VMEM((T * NP, 3 * H), bf16)],
    )(
        xT, W_ih.T.astype(bf16), W_hh.T.astype(bf16),
        brz, bihn, bhhn,
        A, flat_p, emb_p,
        Wf, bf.reshape(1, -1),
        Wl1.astype(bf16), bl1.reshape(1, -1), Wr1.astype(bf16),
        Wl2.astype(bf16), bl2.reshape(1, -1), Wr2.astype(bf16),
        Wo, bo.reshape(1, 1),
    )
    return out[:N, 0]
